# Initial kernel scaffold; baseline (speedup 1.0000x reference)
import jax
import jax.numpy as jnp
from jax.experimental import pallas as pl


def kernel(idx, alpha):
    raise NotImplementedError("write your pallas kernel here")


# speedup vs baseline: 127.8637x; 127.8637x over previous
import functools

import jax
import jax.numpy as jnp
from jax import lax
from jax.experimental import pallas as pl
from jax.experimental.pallas import tpu as pltpu
from jax.experimental.pallas import tpu_sc as plsc

N_ROWS = 16384
N_COLS = 200
TOTAL = N_ROWS * N_COLS
NC = 2
NS = 16
NW = NC * NS
PER_W = TOTAL // NW
CHUNK = 12800
NCHUNK = PER_W // CHUNK
LANES = 16
UNROLL = 8


@functools.partial(jax.jit, static_argnums=())
def _gather_sigmoid(alpha, idx_flat):
    mesh = plsc.VectorSubcoreMesh(core_axis_name="c", subcore_axis_name="s")

    @functools.partial(
        pl.kernel,
        mesh=mesh,
        out_type=jax.ShapeDtypeStruct((TOTAL,), jnp.float32),
        scratch_types=[
            pltpu.VMEM((CHUNK,), jnp.int32),
            pltpu.VMEM((CHUNK,), jnp.float32),
            pltpu.SemaphoreType.DMA,
        ],
    )
    def k(alpha_hbm, idx_hbm, out_hbm, idx_v, val_v, sem):
        wid = lax.axis_index("s") * NC + lax.axis_index("c")
        base = wid * PER_W

        def chunk_body(kk, carry):
            off = base + kk * CHUNK
            pltpu.sync_copy(idx_hbm.at[pl.ds(off, CHUNK)], idx_v)
            pltpu.async_copy(alpha_hbm.at[idx_v], val_v, sem).wait()

            def inner(i, c2):
                for u in range(UNROLL):
                    j = (i * UNROLL + u) * LANES
                    x = val_v[pl.ds(j, LANES)]
                    val_v[pl.ds(j, LANES)] = 1.0 / (1.0 + jnp.exp(-x))
                return c2

            lax.fori_loop(0, CHUNK // (LANES * UNROLL), inner, 0)
            pltpu.sync_copy(val_v, out_hbm.at[pl.ds(off, CHUNK)])
            return carry

        lax.fori_loop(0, NCHUNK, chunk_body, 0)

    return k(alpha, idx_flat)


def kernel(idx, alpha):
    flat = idx.reshape(TOTAL)
    out = _gather_sigmoid(alpha, flat)
    return out.reshape(idx.shape)


# speedup vs baseline: 139.4383x; 1.0905x over previous
import functools

import jax
import jax.numpy as jnp
from jax import lax
from jax.experimental import pallas as pl
from jax.experimental.pallas import tpu as pltpu
from jax.experimental.pallas import tpu_sc as plsc

N_ROWS = 16384
N_COLS = 200
TOTAL = N_ROWS * N_COLS
NC = 2
NS = 16
NW = NC * NS
PER_W = TOTAL // NW
CHUNK = 12800
NCHUNK = PER_W // CHUNK
LANES = 16
UNROLL = 8


def _gather_sigmoid(alpha, idx_flat):
    mesh = plsc.VectorSubcoreMesh(core_axis_name="c", subcore_axis_name="s")

    @functools.partial(
        pl.kernel,
        mesh=mesh,
        out_type=jax.ShapeDtypeStruct((TOTAL,), jnp.float32),
        scratch_types=[
            pltpu.VMEM((CHUNK,), jnp.int32),
            pltpu.VMEM((CHUNK,), jnp.int32),
            pltpu.VMEM((CHUNK,), jnp.float32),
            pltpu.VMEM((CHUNK,), jnp.float32),
            pltpu.SemaphoreType.DMA,
            pltpu.SemaphoreType.DMA,
            pltpu.SemaphoreType.DMA,
            pltpu.SemaphoreType.DMA,
            pltpu.SemaphoreType.DMA,
            pltpu.SemaphoreType.DMA,
        ],
    )
    def k(alpha_hbm, idx_hbm, out_hbm,
          idx_v0, idx_v1, val_v0, val_v1,
          si0, si1, sg0, sg1, so0, so1):
        idx_v = [idx_v0, idx_v1]
        val_v = [val_v0, val_v1]
        s_i = [si0, si1]
        s_g = [sg0, sg1]
        s_o = [so0, so1]

        wid = lax.axis_index("s") * NC + lax.axis_index("c")
        base = wid * PER_W

        def idx_copy(c):
            return pltpu.async_copy(
                idx_hbm.at[pl.ds(base + c * CHUNK, CHUNK)], idx_v[c % 2], s_i[c % 2])

        def gather(c):
            return pltpu.async_copy(
                alpha_hbm.at[idx_v[c % 2]], val_v[c % 2], s_g[c % 2])

        def out_copy(c):
            return pltpu.async_copy(
                val_v[c % 2], out_hbm.at[pl.ds(base + c * CHUNK, CHUNK)], s_o[c % 2])

        def sigmoid(buf):
            def inner(i, c2):
                for u in range(UNROLL):
                    j = (i * UNROLL + u) * LANES
                    x = buf[pl.ds(j, LANES)]
                    buf[pl.ds(j, LANES)] = 1.0 / (1.0 + jnp.exp(-x))
                return c2
            lax.fori_loop(0, CHUNK // (LANES * UNROLL), inner, 0)

        h_idx = {0: idx_copy(0)}
        h_idx[0].wait()
        h_g = {0: gather(0)}
        h_idx[1] = idx_copy(1)
        h_out = {}
        for c in range(NCHUNK):
            b = c % 2
            if c + 1 < NCHUNK:
                h_idx[c + 1].wait()
                if c >= 1:
                    h_out[c - 1].wait()
                h_g[c + 1] = gather(c + 1)
            h_g[c].wait()
            if c + 2 < NCHUNK:
                h_idx[c + 2] = idx_copy(c + 2)
            sigmoid(val_v[b])
            h_out[c] = out_copy(c)
        h_out[NCHUNK - 2].wait()
        h_out[NCHUNK - 1].wait()

    return k(alpha, idx_flat)


def kernel(idx, alpha):
    flat = idx.reshape(TOTAL)
    out = _gather_sigmoid(alpha, flat)
    return out.reshape(idx.shape)


# speedup vs baseline: 210.4213x; 1.5091x over previous
import functools

import jax
import jax.numpy as jnp
from jax import lax
from jax.experimental import pallas as pl
from jax.experimental.pallas import tpu as pltpu
from jax.experimental.pallas import tpu_sc as plsc

N_ROWS = 16384
N_COLS = 200
TOTAL = N_ROWS * N_COLS
TABLE = 1_000_000
NC = 2
NS = 16
NW = NC * NS
PER_W = TOTAL // NW
CHUNK = 12800
NCHUNK = PER_W // CHUNK
LANES = 16
UNROLL = 8
PIECE = 8192
T_SLICE = 65536
TAIL = TABLE - 15 * T_SLICE


def _gather_sigmoid(alpha, idx_flat):
    mesh = plsc.VectorSubcoreMesh(core_axis_name="c", subcore_axis_name="s")

    @functools.partial(
        pl.kernel,
        mesh=mesh,
        out_type=jax.ShapeDtypeStruct((TOTAL,), jnp.float32),
        scratch_types=[
            pltpu.VMEM_SHARED((TABLE,), jnp.float32),
            pltpu.VMEM((CHUNK,), jnp.int32),
            pltpu.VMEM((CHUNK,), jnp.int32),
            pltpu.VMEM((CHUNK,), jnp.float32),
            pltpu.VMEM((CHUNK,), jnp.float32),
            pltpu.SemaphoreType.DMA,
            pltpu.SemaphoreType.DMA,
            pltpu.SemaphoreType.DMA,
            pltpu.SemaphoreType.DMA,
            pltpu.SemaphoreType.DMA,
            pltpu.SemaphoreType.DMA,
        ],
    )
    def k(alpha_hbm, idx_hbm, out_hbm,
          table_s,
          idx_v0, idx_v1, val_v0, val_v1,
          si0, si1, sg0, sg1, so0, so1):
        idx_v = [idx_v0, idx_v1]
        val_v = [val_v0, val_v1]
        s_i = [si0, si1]
        s_g = [sg0, sg1]
        s_o = [so0, so1]

        cid = lax.axis_index("c")
        tid = lax.axis_index("s")
        wid = tid * NC + cid
        base = wid * PER_W

        def sigmoid_range(buf, length):
            full = length // (LANES * UNROLL)

            def inner(i, c2):
                for u in range(UNROLL):
                    j = (i * UNROLL + u) * LANES
                    x = buf[pl.ds(j, LANES)]
                    buf[pl.ds(j, LANES)] = 1.0 / (1.0 + jnp.exp(-x))
                return c2

            lax.fori_loop(0, full, inner, 0)
            for r in range(full * LANES * UNROLL, length, LANES):
                x = buf[pl.ds(r, LANES)]
                buf[pl.ds(r, LANES)] = 1.0 / (1.0 + jnp.exp(-x))

        def stage_piece(off, length):
            pltpu.sync_copy(alpha_hbm.at[pl.ds(off, length)],
                            val_v0.at[pl.ds(0, length)])
            sigmoid_range(val_v0, length)
            pltpu.sync_copy(val_v0.at[pl.ds(0, length)],
                            table_s.at[pl.ds(off, length)])

        @pl.when(tid < NS - 1)
        def _():
            for p in range(T_SLICE // PIECE):
                stage_piece(tid * T_SLICE + p * PIECE, PIECE)

        @pl.when(tid == NS - 1)
        def _():
            for p in range(2):
                stage_piece((NS - 1) * T_SLICE + p * PIECE, PIECE)
            stage_piece((NS - 1) * T_SLICE + 2 * PIECE, TAIL - 2 * PIECE)

        plsc.subcore_barrier()

        def idx_copy(c):
            return pltpu.async_copy(
                idx_hbm.at[pl.ds(base + c * CHUNK, CHUNK)], idx_v[c % 2], s_i[c % 2])

        def gather(c):
            return pltpu.async_copy(
                table_s.at[idx_v[c % 2]], val_v[c % 2], s_g[c % 2])

        def out_copy(c):
            return pltpu.async_copy(
                val_v[c % 2], out_hbm.at[pl.ds(base + c * CHUNK, CHUNK)], s_o[c % 2])

        h_idx = {0: idx_copy(0)}
        h_idx[0].wait()
        h_g = {0: gather(0)}
        h_idx[1] = idx_copy(1)
        h_out = {}
        for c in range(NCHUNK):
            b = c % 2
            if c + 1 < NCHUNK:
                h_idx[c + 1].wait()
                if c >= 1:
                    h_out[c - 1].wait()
                h_g[c + 1] = gather(c + 1)
            h_g[c].wait()
            if c + 2 < NCHUNK:
                h_idx[c + 2] = idx_copy(c + 2)
            h_out[c] = out_copy(c)
        h_out[NCHUNK - 2].wait()
        h_out[NCHUNK - 1].wait()

    return k(alpha, idx_flat)


def kernel(idx, alpha):
    flat = idx.reshape(TOTAL)
    out = _gather_sigmoid(alpha, flat)
    return out.reshape(idx.shape)
